# baseline (speedup 1.0000x reference)
import functools

import jax
import jax.numpy as jnp
from jax import lax
from jax.experimental import pallas as pl
from jax.experimental.pallas import tpu as pltpu
from jax.experimental.pallas import tpu_sc as plsc

DEPTH = 1000
ON_VALUE = 1.0
OFF_VALUE = 0.0

NUM_CORES = 2
NUM_SUBCORES = 16
NUM_WORKERS = NUM_CORES * NUM_SUBCORES
LANES = 16

CHUNK_B = 2
F_PAD = 32
D_PAD = 1024


def _one_hot_sc(idx_flat, b_total, f_total):
  batches_per_worker = b_total // NUM_WORKERS
  n_chunks = batches_per_worker // CHUNK_B
  chunk_rows = CHUNK_B * f_total
  rows_per_worker = batches_per_worker * f_total
  n_groups = -(-chunk_rows // LANES)
  payload = f_total * D_PAD
  batch_stride = F_PAD * D_PAD

  mesh = plsc.VectorSubcoreMesh(core_axis_name="c", subcore_axis_name="s")

  @functools.partial(
      pl.kernel,
      mesh=mesh,
      out_type=jax.ShapeDtypeStruct((b_total * batch_stride,), jnp.float32),
      scratch_types=[
          pltpu.VMEM((rows_per_worker,), jnp.int32),
          pltpu.VMEM((CHUNK_B * payload,), jnp.float32),
      ],
      compiler_params=pltpu.CompilerParams(needs_layout_passes=False),
  )
  def k(idx_hbm, out_hbm, idx_v, buf):
    wid = lax.axis_index("s") * NUM_CORES + lax.axis_index("c")
    batch0 = wid * batches_per_worker

    pltpu.sync_copy(idx_hbm.at[pl.ds(batch0 * f_total, rows_per_worker)],
                    idx_v)

    zeros16 = jnp.zeros((LANES,), jnp.float32)

    def zero_body(i, _):
      base = i * (8 * LANES)
      for u in range(8):
        buf[pl.ds(base + u * LANES, LANES)] = zeros16
      return 0

    lax.fori_loop(0, CHUNK_B * payload // (8 * LANES), zero_body, 0)

    lane = lax.iota(jnp.int32, LANES)
    ones16 = jnp.full((LANES,), jnp.float32(ON_VALUE))

    def scatter_chunk(c, val16):
      for g in range(n_groups):
        j = lane + g * LANES
        mask = j < chunk_rows if (g + 1) * LANES > chunk_rows else None
        d = plsc.load_gather(idx_v, [j + c * chunk_rows], mask=mask)
        b = jnp.where(j >= f_total, 1, 0)
        f = j - b * f_total
        plsc.store_scatter(buf, [b * payload + f * D_PAD + d], val16,
                           mask=mask)

    def chunk_body(c, _):
      scatter_chunk(c, ones16)
      for b in range(CHUNK_B):
        pltpu.sync_copy(
            buf.at[pl.ds(b * payload, payload)],
            out_hbm.at[pl.ds((batch0 + c * CHUNK_B + b) * batch_stride,
                             payload)])
      scatter_chunk(c, zeros16)
      return 0

    lax.fori_loop(0, n_chunks, chunk_body, 0)

  return k(idx_flat)


@jax.jit
def kernel(indices):
  b, f = indices.shape
  out = _one_hot_sc(indices.reshape(-1), b, f)
  out = out.reshape(b, F_PAD, D_PAD)
  return lax.slice(out, (0, 0, 0), (b, f, DEPTH))


# speedup vs baseline: 1.1849x; 1.1849x over previous
import functools

import jax
import jax.numpy as jnp
from jax import lax
from jax.experimental import pallas as pl
from jax.experimental.pallas import tpu as pltpu
from jax.experimental.pallas import tpu_sc as plsc

DEPTH = 1000
ON_VALUE = 1.0
OFF_VALUE = 0.0

NUM_CORES = 2
NUM_SUBCORES = 16
NUM_WORKERS = NUM_CORES * NUM_SUBCORES
LANES = 16

CHUNK_B = 2
F_PAD = 32
D_PAD = 1024


def _one_hot_sc(idx_flat, b_total, f_total):
  batches_per_worker = b_total // NUM_WORKERS
  n_chunks = batches_per_worker // CHUNK_B
  chunk_rows = CHUNK_B * f_total
  rows_per_worker = batches_per_worker * f_total
  n_groups = -(-chunk_rows // LANES)

  mesh = plsc.VectorSubcoreMesh(core_axis_name="c", subcore_axis_name="s")

  @functools.partial(
      pl.kernel,
      mesh=mesh,
      out_type=jax.ShapeDtypeStruct((b_total, F_PAD, D_PAD), jnp.float32),
      scratch_types=[
          pltpu.VMEM((rows_per_worker,), jnp.int32),
          pltpu.VMEM((CHUNK_B, F_PAD, D_PAD), jnp.float32),
      ],
      compiler_params=pltpu.CompilerParams(needs_layout_passes=False),
  )
  def k(idx_hbm, out_hbm, idx_v, buf):
    wid = lax.axis_index("s") * NUM_CORES + lax.axis_index("c")
    batch0 = wid * batches_per_worker

    pltpu.sync_copy(idx_hbm.at[pl.ds(batch0 * f_total, rows_per_worker)],
                    idx_v)

    zeros16 = jnp.zeros((LANES,), jnp.float32)

    def zero_body(i, _):
      for b in range(CHUNK_B):
        for f in range(F_PAD):
          buf[b, f, pl.ds(i * LANES, LANES)] = zeros16
      return 0

    lax.fori_loop(0, D_PAD // LANES, zero_body, 0)

    lane = lax.iota(jnp.int32, LANES)
    ones16 = jnp.full((LANES,), jnp.float32(ON_VALUE))

    def scatter_chunk(c, val16):
      for g in range(n_groups):
        j = lane + g * LANES
        mask = j < chunk_rows if (g + 1) * LANES > chunk_rows else None
        d = plsc.load_gather(idx_v, [j + c * chunk_rows], mask=mask)
        b = jnp.where(j >= f_total, 1, 0)
        f = j - b * f_total
        plsc.store_scatter(buf, [b, f, d], val16, mask=mask)

    def chunk_body(c, _):
      scatter_chunk(c, ones16)
      pltpu.sync_copy(buf, out_hbm.at[pl.ds(batch0 + c * CHUNK_B, CHUNK_B)])
      scatter_chunk(c, zeros16)
      return 0

    lax.fori_loop(0, n_chunks, chunk_body, 0)

  return k(idx_flat)


@jax.jit
def kernel(indices):
  b, f = indices.shape
  out = _one_hot_sc(indices.reshape(-1), b, f)
  sliced = lax.slice(out, (0, 0, 0), (b, f, DEPTH))
  scale = lax.optimization_barrier(jnp.float32(ON_VALUE))
  return sliced * scale


# speedup vs baseline: 1.8979x; 1.6016x over previous
import functools

import jax
import jax.numpy as jnp
from jax import lax
from jax.experimental import pallas as pl
from jax.experimental.pallas import tpu as pltpu
from jax.experimental.pallas import tpu_sc as plsc

DEPTH = 1000
ON_VALUE = 1.0
OFF_VALUE = 0.0

NUM_CORES = 2
NUM_SUBCORES = 16
NUM_WORKERS = NUM_CORES * NUM_SUBCORES
LANES = 16

CHUNK_B = 2
F_PAD = 32
D_PAD = 1024


def _one_hot_sc(idx_flat, b_total, f_total):
  batches_per_worker = b_total // NUM_WORKERS
  n_chunks = batches_per_worker // CHUNK_B
  chunk_rows = CHUNK_B * f_total
  rows_per_worker = batches_per_worker * f_total
  n_groups = -(-chunk_rows // LANES)

  mesh = plsc.VectorSubcoreMesh(core_axis_name="c", subcore_axis_name="s")

  @functools.partial(
      pl.kernel,
      mesh=mesh,
      out_type=jax.ShapeDtypeStruct((b_total, F_PAD, D_PAD), jnp.float32),
      scratch_types=[
          pltpu.VMEM((rows_per_worker,), jnp.int32),
          pltpu.VMEM((CHUNK_B, F_PAD, D_PAD), jnp.float32),
      ],
      compiler_params=pltpu.CompilerParams(needs_layout_passes=False),
  )
  def k(idx_hbm, out_hbm, idx_v, buf):
    wid = lax.axis_index("s") * NUM_CORES + lax.axis_index("c")
    batch0 = wid * batches_per_worker

    pltpu.sync_copy(idx_hbm.at[pl.ds(batch0 * f_total, rows_per_worker)],
                    idx_v)

    zeros16 = jnp.zeros((LANES,), jnp.float32)

    def zero_body(i, _):
      for b in range(CHUNK_B):
        for f in range(F_PAD):
          buf[b, f, pl.ds(i * LANES, LANES)] = zeros16
      return 0

    lax.fori_loop(0, D_PAD // LANES, zero_body, 0)

    lane = lax.iota(jnp.int32, LANES)
    ones16 = jnp.full((LANES,), jnp.float32(ON_VALUE))

    def scatter_chunk(c, val16):
      for g in range(n_groups):
        j = lane + g * LANES
        mask = j < chunk_rows if (g + 1) * LANES > chunk_rows else None
        d = plsc.load_gather(idx_v, [j + c * chunk_rows], mask=mask)
        b = jnp.where(j >= f_total, 1, 0)
        f = j - b * f_total
        plsc.store_scatter(buf, [b, f, d], val16, mask=mask)

    def chunk_body(c, _):
      scatter_chunk(c, ones16)
      pltpu.sync_copy(buf, out_hbm.at[pl.ds(batch0 + c * CHUNK_B, CHUNK_B)])
      scatter_chunk(c, zeros16)
      return 0

    lax.fori_loop(0, n_chunks, chunk_body, 0)

  return k(idx_flat)


@jax.jit
def kernel(indices):
  b, f = indices.shape
  out = _one_hot_sc(indices.reshape(-1), b, f)
  return lax.slice(out, (0, 0, 0), (b, f, DEPTH))
